# Initial kernel scaffold; baseline (speedup 1.0000x reference)
import jax
import jax.numpy as jnp
from jax.experimental import pallas as pl


def kernel(nodes, history_uv, history_r, features, rating_table, W_agg, b_agg, W1, b1):
    raise NotImplementedError("write your pallas kernel here")


# speedup vs baseline: 2.5635x; 2.5635x over previous
import functools

import jax
import jax.numpy as jnp
from jax import lax
from jax.experimental import pallas as pl
from jax.experimental.pallas import tpu as pltpu
from jax.experimental.pallas import tpu_sc as plsc

B = 4096
V = 100000
D = 64
H = 50
R = 5

NC = 2
NS = 16
NW = NC * NS
NB = B // NW
LANES = 16
NVD = D // LANES

_VBLK = 1024
_BBLK = 512


def _proj_body(x_ref, w_ref, o_ref):
    o_ref[...] = jnp.dot(x_ref[...], w_ref[...],
                         preferred_element_type=jnp.float32)


def _project(features, wu):
    grid = (pl.cdiv(V, _VBLK),)
    return pl.pallas_call(
        _proj_body,
        grid=grid,
        in_specs=[pl.BlockSpec((_VBLK, D), lambda i: (i, 0)),
                  pl.BlockSpec((D, D), lambda i: (0, 0))],
        out_specs=pl.BlockSpec((_VBLK, D), lambda i: (i, 0)),
        out_shape=jax.ShapeDtypeStruct((V, D), jnp.float32),
    )(features, wu)


def _rp_body(rt_ref, w_ref, b_ref, o_ref):
    o_ref[...] = jnp.dot(rt_ref[...], w_ref[...],
                         preferred_element_type=jnp.float32) + b_ref[...]


def _rating_proj(rating_table, wr, b_agg):
    return pl.pallas_call(
        _rp_body,
        out_shape=jax.ShapeDtypeStruct((R, D), jnp.float32),
    )(rating_table, wr, b_agg.reshape(1, D))


def _comb_body(s_ref, n_ref, wa_ref, wb_ref, b_ref, o_ref):
    acc = jnp.dot(s_ref[...], wa_ref[...], preferred_element_type=jnp.float32)
    acc += jnp.dot(n_ref[...], wb_ref[...], preferred_element_type=jnp.float32)
    o_ref[...] = jnp.maximum(acc + b_ref[...], 0.0)


def _combine(self_feats, neigh, wa, wb, b1):
    grid = (B // _BBLK,)
    return pl.pallas_call(
        _comb_body,
        grid=grid,
        in_specs=[pl.BlockSpec((_BBLK, D), lambda i: (i, 0)),
                  pl.BlockSpec((_BBLK, D), lambda i: (i, 0)),
                  pl.BlockSpec((D, D), lambda i: (0, 0)),
                  pl.BlockSpec((D, D), lambda i: (0, 0)),
                  pl.BlockSpec((1, D), lambda i: (0, 0))],
        out_specs=pl.BlockSpec((_BBLK, D), lambda i: (i, 0)),
        out_shape=jax.ShapeDtypeStruct((B, D), jnp.float32),
    )(self_feats, neigh, wa, wb, b1.reshape(1, D))


HP = 112
HG = 56


def _sc_gather_agg(nodes, hist, fp, rp, features):
    mesh = plsc.VectorSubcoreMesh(core_axis_name="c", subcore_axis_name="s")

    @functools.partial(
        pl.kernel,
        mesh=mesh,
        compiler_params=pltpu.CompilerParams(use_tc_tiling_on_sc=False),
        out_type=[jax.ShapeDtypeStruct((B, D), jnp.float32),
                  jax.ShapeDtypeStruct((B, D), jnp.float32)],
        scratch_types=[
            pltpu.VMEM((NB,), jnp.int32),
            pltpu.VMEM((NB, HP), jnp.int32),
            pltpu.VMEM((NB, D), jnp.float32),
            pltpu.VMEM((R, D), jnp.float32),
            pltpu.VMEM((HG, D), jnp.float32),
            pltpu.VMEM((HG, D), jnp.float32),
            pltpu.VMEM((NB, D), jnp.float32),
            pltpu.SemaphoreType.DMA,
            pltpu.SemaphoreType.DMA,
            pltpu.SemaphoreType.DMA,
            pltpu.SemaphoreType.DMA,
        ],
    )
    def k(nodes_hbm, hist_hbm, fp_hbm, rp_hbm, feat_hbm,
          neigh_hbm, self_hbm,
          nodes_v, hist_v, self_v, rp_v, rows0, rows1, neigh_v,
          sem_a, sem_c, sem0, sem1):
        wid = lax.axis_index("s") * NC + lax.axis_index("c")
        base = wid * NB

        pltpu.sync_copy(nodes_hbm.at[pl.ds(base, NB)], nodes_v)
        cp_h = pltpu.async_copy(hist_hbm.at[nodes_v], hist_v, sem_a)
        cp_sf = pltpu.async_copy(feat_hbm.at[nodes_v], self_v, sem_c)
        pltpu.sync_copy(rp_hbm, rp_v)
        cp_h.wait()
        cp_sf.wait()

        def compute(n, rows_ref):
            hrv = (hist_v[n, pl.ds(56, LANES)],
                   hist_v[n, pl.ds(72, LANES)],
                   hist_v[n, pl.ds(88, LANES)],
                   hist_v[n, pl.ds(96, LANES)])
            accs = [jnp.zeros((LANES,), jnp.float32) for _ in range(NVD)]
            for h in range(H):
                if h < 48:
                    r = hrv[h // LANES][h % LANES]
                else:
                    r = hrv[3][h - 40]
                for d in range(NVD):
                    fpv = rows_ref[h, pl.ds(d * LANES, LANES)]
                    rpv = rp_v[r, pl.ds(d * LANES, LANES)]
                    accs[d] = accs[d] + jnp.maximum(fpv + rpv, 0.0)
            inv = jnp.float32(1.0 / H)
            for d in range(NVD):
                neigh_v[n, pl.ds(d * LANES, LANES)] = accs[d] * inv

        def hu_idx(n):
            return hist_v.at[n, pl.ds(0, HG)]

        pltpu.async_copy(fp_hbm.at[hu_idx(0)], rows0, sem0)

        def body2(i, carry):
            n0 = 2 * i
            n1 = n0 + 1

            pltpu.async_copy(fp_hbm.at[hu_idx(n1)], rows1, sem1)

            pltpu.make_async_copy(fp_hbm.at[hu_idx(n0)], rows0, sem0).wait()
            compute(n0, rows0)

            @pl.when(n0 + 2 < NB)
            def _():
                pltpu.async_copy(fp_hbm.at[hu_idx(n0 + 2)], rows0, sem0)

            pltpu.make_async_copy(fp_hbm.at[hu_idx(n1)], rows1, sem1).wait()
            compute(n1, rows1)
            return carry

        lax.fori_loop(0, NB // 2, body2, 0)

        pltpu.sync_copy(neigh_v, neigh_hbm.at[pl.ds(base, NB)])
        pltpu.sync_copy(self_v, self_hbm.at[pl.ds(base, NB)])

    return k(nodes, hist, fp, rp, features)


def kernel(nodes, history_uv, history_r, features, rating_table,
           W_agg, b_agg, W1, b1):
    nodes = nodes.astype(jnp.int32)
    history_uv = history_uv.astype(jnp.int32)
    history_r = history_r.astype(jnp.int32)
    zpad = jnp.zeros((V, HP // 2 - H), jnp.int32)
    hist = jnp.concatenate([history_uv, zpad, history_r, zpad], axis=1)
    wu = W_agg[:D]
    wr = W_agg[D:]
    fp = _project(features, wu)
    rp = _rating_proj(rating_table, wr, b_agg)
    neigh, self_feats = _sc_gather_agg(nodes, hist, fp, rp, features)
    return _combine(self_feats, neigh, W1[:D], W1[D:], b1)
